# baseline (speedup 1.0000x reference)
import functools

import jax
import jax.numpy as jnp
from jax import lax
from jax.experimental import pallas as pl
from jax.experimental.pallas import tpu as pltpu
from jax.experimental.pallas import tpu_sc as plsc

_NC = 2
_NS = 16
_NW = _NC * _NS

_CHUNK = 128
_GRP = 2
_NSLOT = 4
_LAG = 2


@functools.partial(jax.jit, static_argnames=("n_chunks",))
def _lookup(x1d, table, n_chunks):
    d = table.shape[1]
    b_total = x1d.shape[0]
    grp_rows = _GRP * _CHUNK
    n_groups = n_chunks // _GRP
    assert n_chunks % _GRP == 0
    assert n_groups > _NSLOT and (n_groups - _NSLOT) % _NSLOT == 0
    mesh = plsc.VectorSubcoreMesh(core_axis_name="c", subcore_axis_name="s")

    @functools.partial(
        pl.kernel,
        mesh=mesh,
        compiler_params=pltpu.CompilerParams(use_tc_tiling_on_sc=False),
        out_type=jax.ShapeDtypeStruct((b_total, d), jnp.float32),
        scratch_types=[
            pltpu.VMEM((n_chunks * _CHUNK,), jnp.int32),
            pltpu.VMEM((_NSLOT, grp_rows, d), jnp.float32),
        ]
        + [pltpu.SemaphoreType.DMA] * (2 * _NSLOT),
    )
    def k(idx_hbm, table_hbm, out_hbm, idx_v, rows_v, *sems):
        gs = sems[:_NSLOT]
        ws = sems[_NSLOT:]
        wid = lax.axis_index("s") * _NC + lax.axis_index("c")
        base_row = wid * n_chunks * _CHUNK
        pltpu.sync_copy(idx_hbm.at[pl.ds(base_row, n_chunks * _CHUNK)], idx_v)

        def fire_gather(g, s):
            for b in range(_GRP):
                pltpu.async_copy(
                    table_hbm.at[idx_v.at[pl.ds((g * _GRP + b) * _CHUNK, _CHUNK)]],
                    rows_v.at[s, pl.ds(b * _CHUNK, _CHUNK)],
                    gs[s],
                )

        def wait_gather(s):
            for _ in range(_GRP):
                pltpu.make_async_copy(
                    table_hbm.at[pl.ds(0, _CHUNK)],
                    rows_v.at[s, pl.ds(0, _CHUNK)],
                    gs[s],
                ).wait()

        def fire_write(g, s):
            pltpu.async_copy(
                rows_v.at[s],
                out_hbm.at[pl.ds(base_row + g * grp_rows, grp_rows)],
                ws[s],
            )

        def wait_write(s):
            pltpu.make_async_copy(
                rows_v.at[s],
                out_hbm.at[pl.ds(0, grp_rows)],
                ws[s],
            ).wait()

        for g in range(_NSLOT):
            fire_gather(g, g)
            if g >= _LAG:
                wait_gather(g - _LAG)
                fire_write(g - _LAG, g - _LAG)

        def body(i, _):
            q = _NSLOT + i * _NSLOT
            for j in range(_NSLOT):
                g = q + j
                wait_write(j)
                fire_gather(g, j)
                s2 = (j + _NSLOT - _LAG) % _NSLOT
                wait_gather(s2)
                fire_write(g - _LAG, s2)
            return _

        lax.fori_loop(0, (n_groups - _NSLOT) // _NSLOT, body, None)

        for g in range(n_groups - _LAG, n_groups):
            s = g % _NSLOT
            wait_gather(s)
            fire_write(g, s)
        for s in range(_NSLOT):
            wait_write(s)

    return k(x1d, table)


def kernel(x, embedding_table):
    b, s = x.shape
    n_total = b * s
    assert n_total % (_NW * _CHUNK) == 0
    n_chunks = n_total // (_NW * _CHUNK)
    x1d = x.reshape(n_total).astype(jnp.int32)
    out = _lookup(x1d, embedding_table, n_chunks)
    return out.reshape(b, s, embedding_table.shape[1])


# speedup vs baseline: 1.0007x; 1.0007x over previous
import functools

import jax
import jax.numpy as jnp
from jax import lax
from jax.experimental import pallas as pl
from jax.experimental.pallas import tpu as pltpu
from jax.experimental.pallas import tpu_sc as plsc

_NC = 2
_NS = 16
_NW = _NC * _NS

_CHUNK = 40
_NSLOT = 4
_LAG = 2


@functools.partial(jax.jit, static_argnames=("rows_per_w",))
def _lookup(x, table, rows_per_w):
    batch, seq = x.shape
    d = table.shape[1]
    n_sub = seq // _CHUNK
    n_groups = rows_per_w
    assert seq % _CHUNK == 0
    assert n_groups > _NSLOT and (n_groups - _NSLOT) % _NSLOT == 0
    mesh = plsc.VectorSubcoreMesh(core_axis_name="c", subcore_axis_name="s")

    @functools.partial(
        pl.kernel,
        mesh=mesh,
        compiler_params=pltpu.CompilerParams(use_tc_tiling_on_sc=False),
        out_type=jax.ShapeDtypeStruct((batch, seq, d), jnp.float32),
        scratch_types=[
            pltpu.VMEM((rows_per_w, seq), jnp.int32),
            pltpu.VMEM((_NSLOT, seq, d), jnp.float32),
        ]
        + [pltpu.SemaphoreType.DMA] * (2 * _NSLOT),
    )
    def k(idx_hbm, table_hbm, out_hbm, idx_v, rows_v, *sems):
        gs = sems[:_NSLOT]
        ws = sems[_NSLOT:]
        wid = lax.axis_index("s") * _NC + lax.axis_index("c")
        base = wid * rows_per_w
        pltpu.sync_copy(idx_hbm.at[pl.ds(base, rows_per_w)], idx_v)

        def fire_gather(g, s):
            for b in range(n_sub):
                pltpu.async_copy(
                    table_hbm.at[idx_v.at[g, pl.ds(b * _CHUNK, _CHUNK)]],
                    rows_v.at[s, pl.ds(b * _CHUNK, _CHUNK)],
                    gs[s],
                )

        def wait_gather(s):
            for _ in range(n_sub):
                pltpu.make_async_copy(
                    table_hbm.at[pl.ds(0, _CHUNK)],
                    rows_v.at[s, pl.ds(0, _CHUNK)],
                    gs[s],
                ).wait()

        def fire_write(g, s):
            pltpu.async_copy(rows_v.at[s], out_hbm.at[base + g], ws[s])

        def wait_write(s):
            pltpu.make_async_copy(rows_v.at[s], out_hbm.at[0], ws[s]).wait()

        for g in range(_NSLOT):
            fire_gather(g, g)
            if g >= _LAG:
                wait_gather(g - _LAG)
                fire_write(g - _LAG, g - _LAG)

        def body(i, _):
            q = _NSLOT + i * _NSLOT
            for j in range(_NSLOT):
                g = q + j
                wait_write(j)
                fire_gather(g, j)
                s2 = (j + _NSLOT - _LAG) % _NSLOT
                wait_gather(s2)
                fire_write(g - _LAG, s2)
            return _

        lax.fori_loop(0, (n_groups - _NSLOT) // _NSLOT, body, None)

        for g in range(n_groups - _LAG, n_groups):
            s = g % _NSLOT
            wait_gather(s)
            fire_write(g, s)
        for s in range(_NSLOT):
            wait_write(s)

    return k(x, table)


def kernel(x, embedding_table):
    b, s = x.shape
    assert b % _NW == 0
    return _lookup(x.astype(jnp.int32), embedding_table, b // _NW)
